# initial kernel scaffold (unmeasured)
import jax
import jax.numpy as jnp
from jax import lax
from jax.experimental import pallas as pl
from jax.experimental.pallas import tpu as pltpu


def kernel(
    x,
):
    def body(*refs):
        pass

    out_shape = jax.ShapeDtypeStruct(..., jnp.float32)
    return pl.pallas_call(body, out_shape=out_shape)(...)


# baseline (device time: 19769 ns/iter reference)
import jax
import jax.numpy as jnp
from jax import lax
from jax.experimental import pallas as pl
from jax.experimental.pallas import tpu as pltpu


def kernel(x):
    m, n = x.shape[2], x.shape[3]

    def body(x_ref, out_ref, send_x, recv_x, send_y, recv_y, sems):
        my_x = lax.axis_index("x")
        my_y = lax.axis_index("y")

        barrier_sem = pltpu.get_barrier_semaphore()
        pl.semaphore_signal(
            barrier_sem, inc=1,
            device_id=(1 - my_x, my_y), device_id_type=pl.DeviceIdType.MESH,
        )
        pl.semaphore_signal(
            barrier_sem, inc=1,
            device_id=(my_x, 1 - my_y), device_id_type=pl.DeviceIdType.MESH,
        )
        pl.semaphore_wait(barrier_sem, 2)

        send_x[...] = x_ref[0, 0, :, :].astype(jnp.bfloat16)
        rdma_x = pltpu.make_async_remote_copy(
            src_ref=send_x,
            dst_ref=recv_x,
            send_sem=sems.at[0],
            recv_sem=sems.at[1],
            device_id=(1 - my_x, my_y),
            device_id_type=pl.DeviceIdType.MESH,
        )
        rdma_x.start()
        rdma_x.wait()

        partial = x_ref[0, 0, :, :] + recv_x[...].astype(jnp.float32)

        send_y[...] = partial.astype(jnp.bfloat16)
        rdma_y = pltpu.make_async_remote_copy(
            src_ref=send_y,
            dst_ref=recv_y,
            send_sem=sems.at[2],
            recv_sem=sems.at[3],
            device_id=(my_x, 1 - my_y),
            device_id_type=pl.DeviceIdType.MESH,
        )
        rdma_y.start()
        rdma_y.wait()

        out_ref[...] = partial + recv_y[...].astype(jnp.float32)

    return pl.pallas_call(
        body,
        out_shape=jax.ShapeDtypeStruct((m, n), jnp.float32),
        in_specs=[pl.BlockSpec(memory_space=pltpu.VMEM)],
        out_specs=pl.BlockSpec(memory_space=pltpu.VMEM),
        scratch_shapes=[
            pltpu.VMEM((m, n), jnp.bfloat16),
            pltpu.VMEM((m, n), jnp.bfloat16),
            pltpu.VMEM((m, n), jnp.bfloat16),
            pltpu.VMEM((m, n), jnp.bfloat16),
            pltpu.SemaphoreType.DMA((4,)),
        ],
        compiler_params=pltpu.CompilerParams(collective_id=0),
    )(x)


# device time: 14181 ns/iter; 1.3940x vs baseline; 1.3940x over previous
import jax
import jax.numpy as jnp
from jax import lax
from jax.experimental import pallas as pl
from jax.experimental.pallas import tpu as pltpu


def kernel(x):
    m, n = x.shape[2], x.shape[3]
    h = m // 2

    def body(x_ref, out_ref, send_a, recv_a, send_b, recv_b,
             send_a2, recv_a2, send_b2, recv_b2, sems):
        my_x = lax.axis_index("x")
        my_y = lax.axis_index("y")
        x_peer = (1 - my_x, my_y)
        y_peer = (my_x, 1 - my_y)

        barrier_sem = pltpu.get_barrier_semaphore()
        for peer in (x_peer, y_peer):
            pl.semaphore_signal(
                barrier_sem, inc=1,
                device_id=peer, device_id_type=pl.DeviceIdType.MESH,
            )
        pl.semaphore_wait(barrier_sem, 2)

        send_a[...] = x_ref[0, 0, :h, :].astype(jnp.bfloat16)
        rdma_a = pltpu.make_async_remote_copy(
            src_ref=send_a, dst_ref=recv_a,
            send_sem=sems.at[0], recv_sem=sems.at[1],
            device_id=x_peer, device_id_type=pl.DeviceIdType.MESH,
        )
        rdma_a.start()
        send_b[...] = x_ref[0, 0, h:, :].astype(jnp.bfloat16)
        rdma_b = pltpu.make_async_remote_copy(
            src_ref=send_b, dst_ref=recv_b,
            send_sem=sems.at[2], recv_sem=sems.at[3],
            device_id=y_peer, device_id_type=pl.DeviceIdType.MESH,
        )
        rdma_b.start()

        rdma_a.wait()
        a1 = x_ref[0, 0, :h, :] + recv_a[...].astype(jnp.float32)
        send_a2[...] = a1.astype(jnp.bfloat16)
        rdma_a2 = pltpu.make_async_remote_copy(
            src_ref=send_a2, dst_ref=recv_a2,
            send_sem=sems.at[4], recv_sem=sems.at[5],
            device_id=y_peer, device_id_type=pl.DeviceIdType.MESH,
        )
        rdma_a2.start()

        rdma_b.wait()
        b1 = x_ref[0, 0, h:, :] + recv_b[...].astype(jnp.float32)
        send_b2[...] = b1.astype(jnp.bfloat16)
        rdma_b2 = pltpu.make_async_remote_copy(
            src_ref=send_b2, dst_ref=recv_b2,
            send_sem=sems.at[6], recv_sem=sems.at[7],
            device_id=x_peer, device_id_type=pl.DeviceIdType.MESH,
        )
        rdma_b2.start()

        rdma_a2.wait()
        out_ref[:h, :] = a1 + recv_a2[...].astype(jnp.float32)
        rdma_b2.wait()
        out_ref[h:, :] = b1 + recv_b2[...].astype(jnp.float32)

    half = lambda: pltpu.VMEM((h, n), jnp.bfloat16)
    return pl.pallas_call(
        body,
        out_shape=jax.ShapeDtypeStruct((m, n), jnp.float32),
        in_specs=[pl.BlockSpec(memory_space=pltpu.VMEM)],
        out_specs=pl.BlockSpec(memory_space=pltpu.VMEM),
        scratch_shapes=[
            half(), half(),
            half(), half(),
            half(), half(),
            half(), half(),
            pltpu.SemaphoreType.DMA((8,)),
        ],
        compiler_params=pltpu.CompilerParams(collective_id=0),
    )(x)


# device time: 12890 ns/iter; 1.5337x vs baseline; 1.1002x over previous
import jax
import jax.numpy as jnp
from jax import lax
from jax.experimental import pallas as pl
from jax.experimental.pallas import tpu as pltpu

NCHUNK = 4


def kernel(x):
    m, n = x.shape[2], x.shape[3]
    q = m // NCHUNK

    def body(x_ref, out_ref, s1, r1, s2, r2, sems):
        my_x = lax.axis_index("x")
        my_y = lax.axis_index("y")
        x_peer = (1 - my_x, my_y)
        y_peer = (my_x, 1 - my_y)
        peers_r1 = [x_peer, x_peer, y_peer, y_peer]
        peers_r2 = [y_peer, y_peer, x_peer, x_peer]

        barrier_sem = pltpu.get_barrier_semaphore()
        for peer in (x_peer, y_peer):
            pl.semaphore_signal(
                barrier_sem, inc=1,
                device_id=peer, device_id_type=pl.DeviceIdType.MESH,
            )
        for i in range(NCHUNK):
            s1[i] = x_ref[0, 0, i * q:(i + 1) * q, :].astype(jnp.bfloat16)
        pl.semaphore_wait(barrier_sem, 2)

        rdma1 = []
        for i in range(NCHUNK):
            d = pltpu.make_async_remote_copy(
                src_ref=s1.at[i], dst_ref=r1.at[i],
                send_sem=sems.at[2 * i], recv_sem=sems.at[2 * i + 1],
                device_id=peers_r1[i], device_id_type=pl.DeviceIdType.MESH,
            )
            d.start()
            rdma1.append(d)

        order = [0, 2, 1, 3]
        rdma2 = [None] * NCHUNK
        for i in order:
            rdma1[i].wait_recv()
            s2[i] = (
                x_ref[0, 0, i * q:(i + 1) * q, :]
                + r1[i].astype(jnp.float32)
            ).astype(jnp.bfloat16)
            d = pltpu.make_async_remote_copy(
                src_ref=s2.at[i], dst_ref=r2.at[i],
                send_sem=sems.at[8 + 2 * i], recv_sem=sems.at[9 + 2 * i],
                device_id=peers_r2[i], device_id_type=pl.DeviceIdType.MESH,
            )
            d.start()
            rdma2[i] = d

        for i in order:
            rdma2[i].wait_recv()
            out_ref[i * q:(i + 1) * q, :] = (
                x_ref[0, 0, i * q:(i + 1) * q, :]
                + r1[i].astype(jnp.float32)
                + r2[i].astype(jnp.float32)
            )

        for i in range(NCHUNK):
            rdma1[i].wait_send()
            rdma2[i].wait_send()

    buf = lambda: pltpu.VMEM((NCHUNK, q, n), jnp.bfloat16)
    return pl.pallas_call(
        body,
        out_shape=jax.ShapeDtypeStruct((m, n), jnp.float32),
        in_specs=[pl.BlockSpec(memory_space=pltpu.VMEM)],
        out_specs=pl.BlockSpec(memory_space=pltpu.VMEM),
        scratch_shapes=[
            buf(), buf(),
            buf(), buf(),
            pltpu.SemaphoreType.DMA((16,)),
        ],
        compiler_params=pltpu.CompilerParams(collective_id=0),
    )(x)
